# Initial kernel scaffold; baseline (speedup 1.0000x reference)
import jax
import jax.numpy as jnp
from jax.experimental import pallas as pl


def kernel(h, x, edge_index, params):
    raise NotImplementedError("write your pallas kernel here")


# speedup vs baseline: 1.8664x; 1.8664x over previous
import functools

import jax
import jax.numpy as jnp
from jax import lax
from jax.experimental import pallas as pl
from jax.experimental.pallas import tpu as pltpu
from jax.experimental.pallas import tpu_sc as plsc

N = 10000
E = 320000
H = 128
NP = 10240
EP = 327680
NC = 2
NS = 16
NW = NC * NS
CH = 128
ECH = 64
EW = EP // NW
NCHUNK = EW // CH
ENCHUNK = EW // ECH
STRIPE = NP // NS
NBLK = 1280
EBLK = 2048

_MESH = plsc.VectorSubcoreMesh(
    core_axis_name="c", subcore_axis_name="s", num_cores=NC, num_subcores=NS)
_SC_PARAMS = pltpu.CompilerParams(needs_layout_passes=False)


def _wid():
  return lax.axis_index("s") * NC + lax.axis_index("c")


@functools.partial(
    pl.kernel,
    out_type=(
        jax.ShapeDtypeStruct((EP,), jnp.float32),
        jax.ShapeDtypeStruct((EP, 16), jnp.float32),
    ),
    mesh=_MESH,
    compiler_params=_SC_PARAMS,
    scratch_types=[
        pltpu.VMEM((NP,), jnp.float32),
        pltpu.VMEM((NP,), jnp.float32),
        pltpu.VMEM((CH,), jnp.int32),
        pltpu.VMEM((CH,), jnp.int32),
        pltpu.VMEM((CH,), jnp.float32),
        pltpu.VMEM((CH, 16), jnp.float32),
    ],
)
def _sc_geom(x0_hbm, x1_hbm, row_hbm, col_hbm, d_out, rp_out,
             x0v, x1v, rowv, colv, dbuf, rpbuf):
  wid = _wid()
  pltpu.sync_copy(x0_hbm, x0v)
  pltpu.sync_copy(x1_hbm, x1v)
  zf = jnp.zeros((16,), jnp.float32)

  def zbody(i, _):
    rpbuf[i, :] = zf
    return 0
  lax.fori_loop(0, CH, zbody, 0)

  iota16 = lax.iota(jnp.int32, 16)
  zeros16 = iota16 * 0
  ones16 = zeros16 + 1

  def chunk(j, _):
    base = wid * EW + j * CH
    pltpu.sync_copy(row_hbm.at[pl.ds(base, CH)], rowv)
    pltpu.sync_copy(col_hbm.at[pl.ds(base, CH)], colv)
    for g in range(CH // 16):
      sl = pl.ds(g * 16, 16)
      ir = rowv[sl]
      ic = colv[sl]
      dx = plsc.load_gather(x0v, [ir]) - plsc.load_gather(x0v, [ic])
      dy = plsc.load_gather(x1v, [ir]) - plsc.load_gather(x1v, [ic])
      dxx = dx * dx
      dyy = dy * dy
      dbuf[sl] = dxx + dyy
      rows16 = zeros16 + g * 16 + iota16
      plsc.store_scatter(rpbuf, [rows16, zeros16], dxx - dyy)
      plsc.store_scatter(rpbuf, [rows16, ones16], 2.0 * dx * dy)
    pltpu.sync_copy(dbuf, d_out.at[pl.ds(base, CH)])
    pltpu.sync_copy(rpbuf, rp_out.at[pl.ds(base, CH)])
    return 0
  lax.fori_loop(0, NCHUNK, chunk, 0)


def _make_sc_edge(with_cnt, with_r):
  outs = [jax.ShapeDtypeStruct((NC, NP, H), jnp.float32)]
  if with_cnt:
    outs.append(jax.ShapeDtypeStruct((NW, NP * 2), jnp.float32))
  if with_r:
    outs.append(jax.ShapeDtypeStruct((EP, H), jnp.float32))
  scratch = [
      pltpu.VMEM((ECH,), jnp.int32),
      pltpu.VMEM((ECH,), jnp.int32),
      pltpu.VMEM((ECH,), jnp.float32),
      pltpu.VMEM((H,), jnp.float32),
      pltpu.VMEM((ECH, H), jnp.float32),
      pltpu.VMEM((ECH, H), jnp.float32),
      pltpu.VMEM((ECH, H), jnp.float32),
      pltpu.VMEM_SHARED((NP, H), jnp.float32),
      pltpu.SemaphoreType.DMA,
      pltpu.SemaphoreType.DMA,
  ]
  if with_cnt:
    scratch.append(pltpu.VMEM((NP * 2,), jnp.float32))

  def body(a_hbm, b_hbm, row_hbm, col_hbm, d_hbm, wr_hbm, *rest):
    idx = 0
    accr_out = rest[idx]; idx += 1
    if with_cnt:
      cntp_out = rest[idx]; idx += 1
    if with_r:
      r_out = rest[idx]; idx += 1
    rowv, colv, dv, wrv, rA, rB, rbuf, accr_sh, semA, semB = rest[idx:idx + 10]
    idx += 10
    if with_cnt:
      cacc = rest[idx]; idx += 1

    c = lax.axis_index("c")
    s = lax.axis_index("s")
    wid = _wid()
    zf = jnp.zeros((16,), jnp.float32)

    def zbody(i, _):
      for g in range(H // 16):
        rbuf[i, pl.ds(g * 16, 16)] = zf
      return 0
    lax.fori_loop(0, ECH, zbody, 0)
    for t in range(STRIPE // ECH):
      pltpu.sync_copy(rbuf, accr_sh.at[pl.ds(s * STRIPE + t * ECH, ECH)])
    if with_cnt:
      def cz(i, _):
        cacc[pl.ds(i * 16, 16)] = zf
        return 0
      lax.fori_loop(0, NP * 2 // 16, cz, 0)
    plsc.subcore_barrier()

    pltpu.sync_copy(wr_hbm, wrv)
    wr_parts = [wrv[pl.ds(g * 16, 16)] for g in range(H // 16)]
    zeros16 = lax.iota(jnp.int32, 16) * 0
    onesf = zeros16.astype(jnp.float32) + 1.0

    def chunk(j, _):
      base = wid * EW + j * ECH
      pltpu.sync_copy(row_hbm.at[pl.ds(base, ECH)], rowv)
      pltpu.sync_copy(col_hbm.at[pl.ds(base, ECH)], colv)
      pltpu.sync_copy(d_hbm.at[pl.ds(base, ECH)], dv)
      cpA = pltpu.async_copy(a_hbm.at[rowv], rA, semA)
      cpB = pltpu.async_copy(b_hbm.at[colv], rB, semB)
      cpA.wait()
      cpB.wait()

      def ebody(e, _):
        de = plsc.load_gather(dv, [zeros16 + e])
        for g in range(H // 16):
          sl = pl.ds(g * 16, 16)
          val = rA[e, sl] + rB[e, sl] + de * wr_parts[g]
          rbuf[e, sl] = jnp.maximum(val, 0.0)
        return 0
      lax.fori_loop(0, ECH, ebody, 0)

      pltpu.sync_copy(rbuf, accr_sh.at[rowv], add=True)
      if with_cnt:
        for g in range(ECH // 16):
          plsc.addupdate_scatter(cacc, [rowv[pl.ds(g * 16, 16)] * 2], onesf)
      if with_r:
        pltpu.sync_copy(rbuf, r_out.at[pl.ds(base, ECH)])
      plsc.subcore_barrier()
      return 0
    lax.fori_loop(0, ENCHUNK, chunk, 0)

    plsc.subcore_barrier()

    def cpout(t, _):
      sl2 = pl.ds(s * STRIPE + t * ECH, ECH)
      pltpu.sync_copy(accr_sh.at[sl2], accr_out.at[c, sl2])
      return 0
    lax.fori_loop(0, STRIPE // ECH, cpout, 0)
    if with_cnt:
      pltpu.sync_copy(cacc, cntp_out.at[wid])

  return pl.kernel(body, out_type=tuple(outs), mesh=_MESH,
                   compiler_params=_SC_PARAMS, scratch_types=scratch)


_sc_edge_first = _make_sc_edge(True, False)
_sc_edge_mid = _make_sc_edge(False, False)
_sc_edge_last = _make_sc_edge(False, True)


@functools.partial(
    pl.kernel,
    out_type=jax.ShapeDtypeStruct((NW, NP * 2), jnp.float32),
    mesh=_MESH,
    compiler_params=_SC_PARAMS,
    scratch_types=[
        pltpu.VMEM((CH,), jnp.int32),
        pltpu.VMEM((CH, 16), jnp.float32),
        pltpu.VMEM((NP * 2,), jnp.float32),
    ],
)
def _sc_srows(srows_hbm, row_hbm, sacc_out, rowv, srcv, acc):
  wid = _wid()
  zf = jnp.zeros((16,), jnp.float32)
  iota16 = lax.iota(jnp.int32, 16)
  zeros16 = iota16 * 0
  ones16 = zeros16 + 1

  def zb(i, _):
    acc[pl.ds(i * 16, 16)] = zf
    return 0
  lax.fori_loop(0, NP * 2 // 16, zb, 0)

  def chunk(j, _):
    base = wid * EW + j * CH
    pltpu.sync_copy(row_hbm.at[pl.ds(base, CH)], rowv)
    pltpu.sync_copy(srows_hbm.at[pl.ds(base, CH)], srcv)
    for g in range(CH // 16):
      sl = pl.ds(g * 16, 16)
      ir = rowv[sl]
      rows16 = zeros16 + g * 16 + iota16
      v0 = plsc.load_gather(srcv, [rows16, zeros16])
      v1 = plsc.load_gather(srcv, [rows16, ones16])
      plsc.addupdate_scatter(acc, [ir * 2], v0)
      plsc.addupdate_scatter(acc, [ir * 2 + 1], v1)
    return 0
  lax.fori_loop(0, NCHUNK, chunk, 0)
  pltpu.sync_copy(acc, sacc_out.at[wid])


def _dot(a, b):
  return jnp.dot(a, b, preferred_element_type=jnp.float32)


_full128 = pl.BlockSpec((H, H), lambda i: (0, 0))
_bias = pl.BlockSpec((1, H), lambda i: (0, 0))
_nodeblk = pl.BlockSpec((NBLK, H), lambda i: (i, 0))
_accblk = pl.BlockSpec((NC, NBLK, H), lambda i: (0, i, 0))
_cntblk = pl.BlockSpec((NBLK, 2), lambda i: (i, 0))
_saccblk = pl.BlockSpec((NBLK, 2), lambda i: (i, 0))


def _tc_pre_body(h_ref, wemb_ref, bemb_ref, wa_ref, ba_ref, wb_ref,
                 h0_ref, a_ref, b_ref):
  h0 = _dot(h_ref[...], wemb_ref[...]) + bemb_ref[...]
  h0_ref[...] = h0
  a_ref[...] = _dot(h0, wa_ref[...]) + ba_ref[...]
  b_ref[...] = _dot(h0, wb_ref[...])


_tc_pre = pl.pallas_call(
    _tc_pre_body,
    grid=(NP // NBLK,),
    in_specs=[_nodeblk, _full128, _bias, _full128, _bias, _full128],
    out_specs=[_nodeblk, _nodeblk, _nodeblk],
    out_shape=[jax.ShapeDtypeStruct((NP, H), jnp.float32)] * 3,
)


def _make_tc_node(first, last):
  def body(*refs):
    idx = 0
    h_ref = refs[idx]; idx += 1
    accr_ref = refs[idx]; idx += 1
    cnt_ref = refs[idx]; idx += 1
    we2, be2, wn1a, wn1b, bn1, wn2, bn2 = refs[idx:idx + 7]; idx += 7
    if last:
      wv1, bv1, wv2, bv2 = refs[idx:idx + 4]; idx += 4
    else:
      wa, ba, wb = refs[idx:idx + 3]; idx += 3
    hn_ref = refs[idx]; idx += 1
    if last:
      vpre_ref = refs[idx]; idx += 1
    else:
      a_ref, b_ref = refs[idx:idx + 2]; idx += 2
    cnt = cnt_ref[...][:, 0:1]
    h = h_ref[...]
    seg = accr_ref[0] + accr_ref[1]
    m = _dot(seg, we2[...]) + cnt * be2[...]
    u = jax.nn.relu(_dot(h, wn1a[...]) + _dot(m, wn1b[...]) + bn1[...])
    hn = h + _dot(u, wn2[...]) + bn2[...]
    hn_ref[...] = hn
    if last:
      vp = jax.nn.relu(_dot(hn, wv1[...]) + bv1[...])
      vpre_ref[...] = _dot(vp, wv2[...]) + bv2[...]
    else:
      a_ref[...] = _dot(hn, wa[...]) + ba[...]
      b_ref[...] = _dot(hn, wb[...])

  in_specs = [_nodeblk, _accblk, _cntblk,
              _full128, _bias, _full128, _full128, _bias, _full128, _bias]
  if last:
    in_specs += [_full128, _bias, _full128, _bias]
  else:
    in_specs += [_full128, _bias, _full128]
  out_specs = [_nodeblk]
  out_shape = [jax.ShapeDtypeStruct((NP, H), jnp.float32)]
  if last:
    out_specs += [_nodeblk]
    out_shape += [jax.ShapeDtypeStruct((NP, H), jnp.float32)]
  else:
    out_specs += [_nodeblk, _nodeblk]
    out_shape += [jax.ShapeDtypeStruct((NP, H), jnp.float32)] * 2
  return pl.pallas_call(body, grid=(NP // NBLK,), in_specs=in_specs,
                        out_specs=out_specs, out_shape=out_shape)


_tc_node_mid = _make_tc_node(False, False)
_tc_node_last = _make_tc_node(False, True)


def _tc_red32_body(x_ref, o_ref):
  o_ref[...] = jnp.sum(x_ref[...], axis=0)


_tc_red32 = pl.pallas_call(
    _tc_red32_body,
    grid=(1,),
    in_specs=[pl.BlockSpec((NW, NP * 2), lambda i: (0, 0))],
    out_specs=pl.BlockSpec((NP * 2,), lambda i: (0,)),
    out_shape=jax.ShapeDtypeStruct((NP * 2,), jnp.float32),
)


def _tc_gate_body(r_ref, rp_ref, we2_ref, wc1_ref, bec_ref, wc2_ref, bc2_ref,
                  srows_ref):
  wec = _dot(we2_ref[...], wc1_ref[...])
  t = jax.nn.relu(_dot(r_ref[...], wec) + bec_ref[...])
  c = jnp.sum(t * wc2_ref[...], axis=-1, keepdims=True) + bc2_ref[:, 0:1]
  srows_ref[...] = rp_ref[...] * c


_tc_gate = pl.pallas_call(
    _tc_gate_body,
    grid=(EP // EBLK,),
    in_specs=[pl.BlockSpec((EBLK, H), lambda i: (i, 0)),
              pl.BlockSpec((EBLK, 16), lambda i: (i, 0)),
              _full128, _full128, _bias, _bias, _bias],
    out_specs=pl.BlockSpec((EBLK, 16), lambda i: (i, 0)),
    out_shape=jax.ShapeDtypeStruct((EP, 16), jnp.float32),
)


def _tc_final_body(vpre_ref, sacc_ref, cnt_ref, v_ref):
  s2 = sacc_ref[...]
  cnt = cnt_ref[...][:, 0:1]
  v2 = vpre_ref[...][:, 0:2] + s2 / jnp.maximum(cnt, 1.0)
  nrm = jnp.sqrt(jnp.sum(v2 * v2, axis=-1, keepdims=True))
  v2 = v2 / jnp.maximum(nrm, 1e-12)
  v_ref[...] = jnp.concatenate(
      [v2, jnp.zeros((v2.shape[0], H - 2), jnp.float32)], axis=-1)


_tc_final = pl.pallas_call(
    _tc_final_body,
    grid=(NP // NBLK,),
    in_specs=[_nodeblk, _saccblk, _cntblk],
    out_specs=_nodeblk,
    out_shape=jax.ShapeDtypeStruct((NP, H), jnp.float32),
)


def kernel(h, x, edge_index, params):
  p = params
  f32 = jnp.float32
  hp = jnp.pad(h.astype(f32), ((0, NP - N), (0, 0)))
  x0 = jnp.pad(x[:, 0].astype(f32), (0, NP - N))
  x1 = jnp.pad(x[:, 1].astype(f32), (0, NP - N))
  row = jnp.pad(edge_index[0], (0, EP - E), constant_values=N)
  col = jnp.pad(edge_index[1], (0, EP - E), constant_values=N)

  we1a = p['W_e1'][:H]
  we1b = p['W_e1'][H:2 * H]
  wr = p['W_e1'][2 * H]
  bemb = p['b_emb'].reshape(1, H)
  be1 = p['b_e1'].reshape(1, H)
  be2 = p['b_e2'].reshape(1, H)
  wn1a = p['W_n1'][:H]
  wn1b = p['W_n1'][H:2 * H]
  bn1 = p['b_n1'].reshape(1, H)
  bn2 = p['b_n2'].reshape(1, H)
  bec = (p['b_e2'] @ p['W_c1'] + p['b_c1']).reshape(1, H)
  wc2 = p['W_c2'][:, 0].reshape(1, H)
  bc2 = jnp.broadcast_to(p['b_c2'].reshape(1, 1), (1, H))
  wv2 = jnp.pad(p['W_v2'], ((0, 0), (0, H - 2)))
  bv2 = jnp.pad(p['b_v2'], (0, H - 2)).reshape(1, H)

  d, rp16 = _sc_geom(x0, x1, row, col)
  h0, a0, b0 = _tc_pre(hp, p['W_emb'], bemb, we1a, be1, we1b)

  accr, cntp = _sc_edge_first(a0, b0, row, col, d, wr)
  cnt2 = _tc_red32(cntp).reshape(NP, 2)
  h1, a1, b1 = _tc_node_mid(
      h0, accr, cnt2, p['W_e2'], be2, wn1a, wn1b, bn1, p['W_n2'], bn2,
      we1a, be1, we1b)

  accr, = _sc_edge_mid(a1, b1, row, col, d, wr)
  h2, a2, b2 = _tc_node_mid(
      h1, accr, cnt2, p['W_e2'], be2, wn1a, wn1b, bn1, p['W_n2'], bn2,
      we1a, be1, we1b)

  accr, = _sc_edge_mid(a2, b2, row, col, d, wr)
  h3, a3, b3 = _tc_node_mid(
      h2, accr, cnt2, p['W_e2'], be2, wn1a, wn1b, bn1, p['W_n2'], bn2,
      we1a, be1, we1b)

  accr, r = _sc_edge_last(a3, b3, row, col, d, wr)
  h4, vpre = _tc_node_last(
      h3, accr, cnt2, p['W_e2'], be2, wn1a, wn1b, bn1, p['W_n2'], bn2,
      p['W_v1'], p['b_v1'].reshape(1, H), wv2, bv2)

  srows = _tc_gate(r, rp16, p['W_e2'], p['W_c1'], bec, wc2, bc2)
  sacc = _sc_srows(srows, row)
  s2 = _tc_red32(sacc).reshape(NP, 2)
  vfull = _tc_final(vpre, s2, cnt2)

  return (h4[:N], x, vfull[:N, :2])


# speedup vs baseline: 2.8621x; 1.5335x over previous
import functools

import jax
import jax.numpy as jnp
from jax import lax
from jax.experimental import pallas as pl
from jax.experimental.pallas import tpu as pltpu
from jax.experimental.pallas import tpu_sc as plsc

N = 10000
E = 320000
H = 128
NP = 10240
EP = 327680
NC = 2
NS = 16
NW = NC * NS
CH = 128
ECH = 64
EW = EP // NW
NCHUNK = EW // CH
ENCHUNK = EW // ECH
STRIPE = NP // NS
NBLK = 1280
EBLK = 2048

_MESH = plsc.VectorSubcoreMesh(
    core_axis_name="c", subcore_axis_name="s", num_cores=NC, num_subcores=NS)
_SC_PARAMS = pltpu.CompilerParams(needs_layout_passes=False)


def _wid():
  return lax.axis_index("s") * NC + lax.axis_index("c")


@functools.partial(
    pl.kernel,
    out_type=(
        jax.ShapeDtypeStruct((EP,), jnp.float32),
        jax.ShapeDtypeStruct((EP, 16), jnp.float32),
    ),
    mesh=_MESH,
    compiler_params=_SC_PARAMS,
    scratch_types=[
        pltpu.VMEM((NP,), jnp.float32),
        pltpu.VMEM((NP,), jnp.float32),
        pltpu.VMEM((CH,), jnp.int32),
        pltpu.VMEM((CH,), jnp.int32),
        pltpu.VMEM((CH,), jnp.float32),
        pltpu.VMEM((CH, 16), jnp.float32),
    ],
)
def _sc_geom(x0_hbm, x1_hbm, row_hbm, col_hbm, d_out, rp_out,
             x0v, x1v, rowv, colv, dbuf, rpbuf):
  wid = _wid()
  pltpu.sync_copy(x0_hbm, x0v)
  pltpu.sync_copy(x1_hbm, x1v)
  zf = jnp.zeros((16,), jnp.float32)

  def zbody(i, _):
    rpbuf[i, :] = zf
    return 0
  lax.fori_loop(0, CH, zbody, 0)

  iota16 = lax.iota(jnp.int32, 16)
  zeros16 = iota16 * 0
  ones16 = zeros16 + 1

  def chunk(j, _):
    base = wid * EW + j * CH
    pltpu.sync_copy(row_hbm.at[pl.ds(base, CH)], rowv)
    pltpu.sync_copy(col_hbm.at[pl.ds(base, CH)], colv)
    for g in range(CH // 16):
      sl = pl.ds(g * 16, 16)
      ir = rowv[sl]
      ic = colv[sl]
      dx = plsc.load_gather(x0v, [ir]) - plsc.load_gather(x0v, [ic])
      dy = plsc.load_gather(x1v, [ir]) - plsc.load_gather(x1v, [ic])
      dxx = dx * dx
      dyy = dy * dy
      dbuf[sl] = dxx + dyy
      rows16 = zeros16 + g * 16 + iota16
      plsc.store_scatter(rpbuf, [rows16, zeros16], dxx - dyy)
      plsc.store_scatter(rpbuf, [rows16, ones16], 2.0 * dx * dy)
    pltpu.sync_copy(dbuf, d_out.at[pl.ds(base, CH)])
    pltpu.sync_copy(rpbuf, rp_out.at[pl.ds(base, CH)])
    return 0
  lax.fori_loop(0, NCHUNK, chunk, 0)


def _make_sc_edge(with_cnt, with_r):
  outs = [jax.ShapeDtypeStruct((NC, NP, H), jnp.float32)]
  if with_cnt:
    outs.append(jax.ShapeDtypeStruct((NW, NP * 2), jnp.float32))
  if with_r:
    outs.append(jax.ShapeDtypeStruct((EP, H), jnp.float32))
  scratch = [
      pltpu.VMEM((3, ECH), jnp.int32),
      pltpu.VMEM((H,), jnp.float32),
      pltpu.VMEM((ECH, H), jnp.float32),
      pltpu.VMEM((ECH, H), jnp.float32),
      pltpu.VMEM((ECH, H), jnp.float32),
      pltpu.VMEM_SHARED((NP, H), jnp.float32),
      pltpu.SemaphoreType.DMA,
      pltpu.SemaphoreType.DMA,
  ]
  if with_cnt:
    scratch.append(pltpu.VMEM((NP * 2,), jnp.float32))

  def body(a_hbm, b_hbm, pk_hbm, wr_hbm, *rest):
    idx = 0
    accr_out = rest[idx]; idx += 1
    if with_cnt:
      cntp_out = rest[idx]; idx += 1
    if with_r:
      r_out = rest[idx]; idx += 1
    pkv, wrv, rA, rB, rbuf, accr_sh, semA, semB = rest[idx:idx + 8]
    idx += 8
    if with_cnt:
      cacc = rest[idx]; idx += 1

    c = lax.axis_index("c")
    s = lax.axis_index("s")
    wid = _wid()
    zf = jnp.zeros((16,), jnp.float32)

    def zbody(i, _):
      for g in range(H // 16):
        rbuf[i, pl.ds(g * 16, 16)] = zf
      return 0
    lax.fori_loop(0, ECH, zbody, 0)
    for t in range(STRIPE // ECH):
      pltpu.sync_copy(rbuf, accr_sh.at[pl.ds(s * STRIPE + t * ECH, ECH)])
    if with_cnt:
      def cz(i, _):
        cacc[pl.ds(i * 16, 16)] = zf
        return 0
      lax.fori_loop(0, NP * 2 // 16, cz, 0)
    plsc.subcore_barrier()

    pltpu.sync_copy(wr_hbm, wrv)
    wr_parts = [wrv[pl.ds(g * 16, 16)] for g in range(H // 16)]
    zeros16 = lax.iota(jnp.int32, 16) * 0
    onesf = zeros16.astype(jnp.float32) + 1.0

    twos16 = zeros16 + 2

    def chunk(j, _):
      base = wid * EW + j * ECH
      gc = wid * ENCHUNK + j
      pltpu.sync_copy(pk_hbm.at[gc], pkv)
      cpA = pltpu.async_copy(a_hbm.at[pkv.at[0]], rA, semA)
      cpB = pltpu.async_copy(b_hbm.at[pkv.at[1]], rB, semB)
      cpA.wait()
      cpB.wait()

      @plsc.parallel_loop(0, ECH, 1, unroll=2)
      def ebody(e):
        de = plsc.bitcast(plsc.load_gather(pkv, [twos16, zeros16 + e]),
                          jnp.float32)
        for g in range(H // 16):
          sl = pl.ds(g * 16, 16)
          val = rA[e, sl] + rB[e, sl] + de * wr_parts[g]
          rbuf[e, sl] = jnp.maximum(val, 0.0)

      pltpu.sync_copy(rbuf, accr_sh.at[pkv.at[0]], add=True)
      if with_cnt:
        for g in range(ECH // 16):
          plsc.addupdate_scatter(cacc, [pkv[0, pl.ds(g * 16, 16)] * 2], onesf)
      if with_r:
        pltpu.sync_copy(rbuf, r_out.at[pl.ds(base, ECH)])
      plsc.subcore_barrier()
      return 0
    lax.fori_loop(0, ENCHUNK, chunk, 0)

    plsc.subcore_barrier()

    def cpout(t, _):
      sl2 = pl.ds(s * STRIPE + t * ECH, ECH)
      pltpu.sync_copy(accr_sh.at[sl2], accr_out.at[c, sl2])
      return 0
    lax.fori_loop(0, STRIPE // ECH, cpout, 0)
    if with_cnt:
      pltpu.sync_copy(cacc, cntp_out.at[wid])

  return pl.kernel(body, out_type=tuple(outs), mesh=_MESH,
                   compiler_params=_SC_PARAMS, scratch_types=scratch)


_sc_edge_first = _make_sc_edge(True, False)
_sc_edge_mid = _make_sc_edge(False, False)
_sc_edge_last = _make_sc_edge(False, True)


@functools.partial(
    pl.kernel,
    out_type=jax.ShapeDtypeStruct((NW, NP * 2), jnp.float32),
    mesh=_MESH,
    compiler_params=_SC_PARAMS,
    scratch_types=[
        pltpu.VMEM((CH,), jnp.int32),
        pltpu.VMEM((CH, 16), jnp.float32),
        pltpu.VMEM((NP * 2,), jnp.float32),
    ],
)
def _sc_srows(srows_hbm, row_hbm, sacc_out, rowv, srcv, acc):
  wid = _wid()
  zf = jnp.zeros((16,), jnp.float32)
  iota16 = lax.iota(jnp.int32, 16)
  zeros16 = iota16 * 0
  ones16 = zeros16 + 1

  def zb(i, _):
    acc[pl.ds(i * 16, 16)] = zf
    return 0
  lax.fori_loop(0, NP * 2 // 16, zb, 0)

  def chunk(j, _):
    base = wid * EW + j * CH
    pltpu.sync_copy(row_hbm.at[pl.ds(base, CH)], rowv)
    pltpu.sync_copy(srows_hbm.at[pl.ds(base, CH)], srcv)
    for g in range(CH // 16):
      sl = pl.ds(g * 16, 16)
      ir = rowv[sl]
      rows16 = zeros16 + g * 16 + iota16
      v0 = plsc.load_gather(srcv, [rows16, zeros16])
      v1 = plsc.load_gather(srcv, [rows16, ones16])
      plsc.addupdate_scatter(acc, [ir * 2], v0)
      plsc.addupdate_scatter(acc, [ir * 2 + 1], v1)
    return 0
  lax.fori_loop(0, NCHUNK, chunk, 0)
  pltpu.sync_copy(acc, sacc_out.at[wid])


def _dot(a, b):
  return jnp.dot(a, b, preferred_element_type=jnp.float32)


_full128 = pl.BlockSpec((H, H), lambda i: (0, 0))
_bias = pl.BlockSpec((1, H), lambda i: (0, 0))
_nodeblk = pl.BlockSpec((NBLK, H), lambda i: (i, 0))
_accblk = pl.BlockSpec((NC, NBLK, H), lambda i: (0, i, 0))
_cntblk = pl.BlockSpec((NBLK, 2), lambda i: (i, 0))
_saccblk = pl.BlockSpec((NBLK, 2), lambda i: (i, 0))


def _tc_pre_body(h_ref, wemb_ref, bemb_ref, wa_ref, ba_ref, wb_ref,
                 h0_ref, a_ref, b_ref):
  h0 = _dot(h_ref[...], wemb_ref[...]) + bemb_ref[...]
  h0_ref[...] = h0
  a_ref[...] = _dot(h0, wa_ref[...]) + ba_ref[...]
  b_ref[...] = _dot(h0, wb_ref[...])


_tc_pre = pl.pallas_call(
    _tc_pre_body,
    grid=(NP // NBLK,),
    in_specs=[_nodeblk, _full128, _bias, _full128, _bias, _full128],
    out_specs=[_nodeblk, _nodeblk, _nodeblk],
    out_shape=[jax.ShapeDtypeStruct((NP, H), jnp.float32)] * 3,
)


def _make_tc_node(first, last):
  def body(*refs):
    idx = 0
    h_ref = refs[idx]; idx += 1
    accr_ref = refs[idx]; idx += 1
    cnt_ref = refs[idx]; idx += 1
    we2, be2, wn1a, wn1b, bn1, wn2, bn2 = refs[idx:idx + 7]; idx += 7
    if last:
      wv1, bv1, wv2, bv2 = refs[idx:idx + 4]; idx += 4
    else:
      wa, ba, wb = refs[idx:idx + 3]; idx += 3
    hn_ref = refs[idx]; idx += 1
    if last:
      vpre_ref = refs[idx]; idx += 1
    else:
      a_ref, b_ref = refs[idx:idx + 2]; idx += 2
    cnt = cnt_ref[...][:, 0:1]
    h = h_ref[...]
    seg = accr_ref[0] + accr_ref[1]
    m = _dot(seg, we2[...]) + cnt * be2[...]
    u = jax.nn.relu(_dot(h, wn1a[...]) + _dot(m, wn1b[...]) + bn1[...])
    hn = h + _dot(u, wn2[...]) + bn2[...]
    hn_ref[...] = hn
    if last:
      vp = jax.nn.relu(_dot(hn, wv1[...]) + bv1[...])
      vpre_ref[...] = _dot(vp, wv2[...]) + bv2[...]
    else:
      a_ref[...] = _dot(hn, wa[...]) + ba[...]
      b_ref[...] = _dot(hn, wb[...])

  in_specs = [_nodeblk, _accblk, _cntblk,
              _full128, _bias, _full128, _full128, _bias, _full128, _bias]
  if last:
    in_specs += [_full128, _bias, _full128, _bias]
  else:
    in_specs += [_full128, _bias, _full128]
  out_specs = [_nodeblk]
  out_shape = [jax.ShapeDtypeStruct((NP, H), jnp.float32)]
  if last:
    out_specs += [_nodeblk]
    out_shape += [jax.ShapeDtypeStruct((NP, H), jnp.float32)]
  else:
    out_specs += [_nodeblk, _nodeblk]
    out_shape += [jax.ShapeDtypeStruct((NP, H), jnp.float32)] * 2
  return pl.pallas_call(body, grid=(NP // NBLK,), in_specs=in_specs,
                        out_specs=out_specs, out_shape=out_shape)


_tc_node_mid = _make_tc_node(False, False)
_tc_node_last = _make_tc_node(False, True)


def _tc_red32_body(x_ref, o_ref):
  o_ref[...] = jnp.sum(x_ref[...], axis=0)


_tc_red32 = pl.pallas_call(
    _tc_red32_body,
    grid=(1,),
    in_specs=[pl.BlockSpec((NW, NP * 2), lambda i: (0, 0))],
    out_specs=pl.BlockSpec((NP * 2,), lambda i: (0,)),
    out_shape=jax.ShapeDtypeStruct((NP * 2,), jnp.float32),
)


def _tc_gate_body(r_ref, rp_ref, we2_ref, wc1_ref, bec_ref, wc2_ref, bc2_ref,
                  srows_ref):
  wec = _dot(we2_ref[...], wc1_ref[...])
  t = jax.nn.relu(_dot(r_ref[...], wec) + bec_ref[...])
  c = jnp.sum(t * wc2_ref[...], axis=-1, keepdims=True) + bc2_ref[:, 0:1]
  srows_ref[...] = rp_ref[...] * c


_tc_gate = pl.pallas_call(
    _tc_gate_body,
    grid=(EP // EBLK,),
    in_specs=[pl.BlockSpec((EBLK, H), lambda i: (i, 0)),
              pl.BlockSpec((EBLK, 16), lambda i: (i, 0)),
              _full128, _full128, _bias, _bias, _bias],
    out_specs=pl.BlockSpec((EBLK, 16), lambda i: (i, 0)),
    out_shape=jax.ShapeDtypeStruct((EP, 16), jnp.float32),
)


def _tc_final_body(vpre_ref, sacc_ref, cnt_ref, v_ref):
  s2 = sacc_ref[...]
  cnt = cnt_ref[...][:, 0:1]
  v2 = vpre_ref[...][:, 0:2] + s2 / jnp.maximum(cnt, 1.0)
  nrm = jnp.sqrt(jnp.sum(v2 * v2, axis=-1, keepdims=True))
  v2 = v2 / jnp.maximum(nrm, 1e-12)
  v_ref[...] = jnp.concatenate(
      [v2, jnp.zeros((v2.shape[0], H - 2), jnp.float32)], axis=-1)


_tc_final = pl.pallas_call(
    _tc_final_body,
    grid=(NP // NBLK,),
    in_specs=[_nodeblk, _saccblk, _cntblk],
    out_specs=_nodeblk,
    out_shape=jax.ShapeDtypeStruct((NP, H), jnp.float32),
)


def kernel(h, x, edge_index, params):
  p = params
  f32 = jnp.float32
  hp = jnp.pad(h.astype(f32), ((0, NP - N), (0, 0)))
  x0 = jnp.pad(x[:, 0].astype(f32), (0, NP - N))
  x1 = jnp.pad(x[:, 1].astype(f32), (0, NP - N))
  row = jnp.pad(edge_index[0], (0, EP - E), constant_values=N)
  col = jnp.pad(edge_index[1], (0, EP - E), constant_values=N)

  we1a = p['W_e1'][:H]
  we1b = p['W_e1'][H:2 * H]
  wr = p['W_e1'][2 * H]
  bemb = p['b_emb'].reshape(1, H)
  be1 = p['b_e1'].reshape(1, H)
  be2 = p['b_e2'].reshape(1, H)
  wn1a = p['W_n1'][:H]
  wn1b = p['W_n1'][H:2 * H]
  bn1 = p['b_n1'].reshape(1, H)
  bn2 = p['b_n2'].reshape(1, H)
  bec = (p['b_e2'] @ p['W_c1'] + p['b_c1']).reshape(1, H)
  wc2 = p['W_c2'][:, 0].reshape(1, H)
  bc2 = jnp.broadcast_to(p['b_c2'].reshape(1, 1), (1, H))
  wv2 = jnp.pad(p['W_v2'], ((0, 0), (0, H - 2)))
  bv2 = jnp.pad(p['b_v2'], (0, H - 2)).reshape(1, H)

  d, rp16 = _sc_geom(x0, x1, row, col)
  pk = jnp.stack([row.reshape(EP // ECH, ECH),
                  col.reshape(EP // ECH, ECH),
                  d.reshape(EP // ECH, ECH).view(jnp.int32)], axis=1)
  h0, a0, b0 = _tc_pre(hp, p['W_emb'], bemb, we1a, be1, we1b)

  accr, cntp = _sc_edge_first(a0, b0, pk, wr)
  cnt2 = _tc_red32(cntp).reshape(NP, 2)
  h1, a1, b1 = _tc_node_mid(
      h0, accr, cnt2, p['W_e2'], be2, wn1a, wn1b, bn1, p['W_n2'], bn2,
      we1a, be1, we1b)

  accr, = _sc_edge_mid(a1, b1, pk, wr)
  h2, a2, b2 = _tc_node_mid(
      h1, accr, cnt2, p['W_e2'], be2, wn1a, wn1b, bn1, p['W_n2'], bn2,
      we1a, be1, we1b)

  accr, = _sc_edge_mid(a2, b2, pk, wr)
  h3, a3, b3 = _tc_node_mid(
      h2, accr, cnt2, p['W_e2'], be2, wn1a, wn1b, bn1, p['W_n2'], bn2,
      we1a, be1, we1b)

  accr, r = _sc_edge_last(a3, b3, pk, wr)
  h4, vpre = _tc_node_last(
      h3, accr, cnt2, p['W_e2'], be2, wn1a, wn1b, bn1, p['W_n2'], bn2,
      p['W_v1'], p['b_v1'].reshape(1, H), wv2, bv2)

  srows = _tc_gate(r, rp16, p['W_e2'], p['W_c1'], bec, wc2, bc2)
  sacc = _sc_srows(srows, row)
  s2 = _tc_red32(sacc).reshape(NP, 2)
  vfull = _tc_final(vpre, s2, cnt2)

  return (h4[:N], x, vfull[:N, :2])


# speedup vs baseline: 2.8633x; 1.0004x over previous
import functools

import jax
import jax.numpy as jnp
from jax import lax
from jax.experimental import pallas as pl
from jax.experimental.pallas import tpu as pltpu
from jax.experimental.pallas import tpu_sc as plsc

N = 10000
E = 320000
H = 128
NP = 10240
EP = 327680
NC = 2
NS = 16
NW = NC * NS
CH = 128
ECH = 64
EW = EP // NW
NCHUNK = EW // CH
ENCHUNK = EW // ECH
STRIPE = NP // NS
NBLK = 1280
EBLK = 2048

_MESH = plsc.VectorSubcoreMesh(
    core_axis_name="c", subcore_axis_name="s", num_cores=NC, num_subcores=NS)
_SC_PARAMS = pltpu.CompilerParams(needs_layout_passes=False)


def _wid():
  return lax.axis_index("s") * NC + lax.axis_index("c")


@functools.partial(
    pl.kernel,
    out_type=(
        jax.ShapeDtypeStruct((EP,), jnp.float32),
        jax.ShapeDtypeStruct((EP, 16), jnp.float32),
    ),
    mesh=_MESH,
    compiler_params=_SC_PARAMS,
    scratch_types=[
        pltpu.VMEM((NP,), jnp.float32),
        pltpu.VMEM((NP,), jnp.float32),
        pltpu.VMEM((CH,), jnp.int32),
        pltpu.VMEM((CH,), jnp.int32),
        pltpu.VMEM((CH,), jnp.float32),
        pltpu.VMEM((CH, 16), jnp.float32),
    ],
)
def _sc_geom(x0_hbm, x1_hbm, row_hbm, col_hbm, d_out, rp_out,
             x0v, x1v, rowv, colv, dbuf, rpbuf):
  wid = _wid()
  pltpu.sync_copy(x0_hbm, x0v)
  pltpu.sync_copy(x1_hbm, x1v)
  zf = jnp.zeros((16,), jnp.float32)

  def zbody(i, _):
    rpbuf[i, :] = zf
    return 0
  lax.fori_loop(0, CH, zbody, 0)

  iota16 = lax.iota(jnp.int32, 16)
  zeros16 = iota16 * 0
  ones16 = zeros16 + 1

  def chunk(j, _):
    base = wid * EW + j * CH
    pltpu.sync_copy(row_hbm.at[pl.ds(base, CH)], rowv)
    pltpu.sync_copy(col_hbm.at[pl.ds(base, CH)], colv)
    for g in range(CH // 16):
      sl = pl.ds(g * 16, 16)
      ir = rowv[sl]
      ic = colv[sl]
      dx = plsc.load_gather(x0v, [ir]) - plsc.load_gather(x0v, [ic])
      dy = plsc.load_gather(x1v, [ir]) - plsc.load_gather(x1v, [ic])
      dxx = dx * dx
      dyy = dy * dy
      dbuf[sl] = dxx + dyy
      rows16 = zeros16 + g * 16 + iota16
      plsc.store_scatter(rpbuf, [rows16, zeros16], dxx - dyy)
      plsc.store_scatter(rpbuf, [rows16, ones16], 2.0 * dx * dy)
    pltpu.sync_copy(dbuf, d_out.at[pl.ds(base, CH)])
    pltpu.sync_copy(rpbuf, rp_out.at[pl.ds(base, CH)])
    return 0
  lax.fori_loop(0, NCHUNK, chunk, 0)


def _make_sc_edge(with_cnt, with_r):
  outs = [jax.ShapeDtypeStruct((NC, NP, H), jnp.float32)]
  if with_cnt:
    outs.append(jax.ShapeDtypeStruct((NW, NP * 2), jnp.float32))
  if with_r:
    outs.append(jax.ShapeDtypeStruct((EP, H), jnp.float32))
  scratch = [
      pltpu.VMEM((3, ECH), jnp.int32),
      pltpu.VMEM((3, ECH), jnp.int32),
      pltpu.VMEM((H,), jnp.float32),
      pltpu.VMEM((ECH, H), jnp.float32),
      pltpu.VMEM((ECH, H), jnp.float32),
      pltpu.VMEM((ECH, H), jnp.float32),
      pltpu.VMEM((ECH, H), jnp.float32),
      pltpu.VMEM_SHARED((NP, H), jnp.float32),
      pltpu.SemaphoreType.DMA,
      pltpu.SemaphoreType.DMA,
  ]
  if with_cnt:
    scratch.append(pltpu.VMEM((NP * 2,), jnp.float32))

  def body(a_hbm, b_hbm, pk_hbm, wr_hbm, *rest):
    idx = 0
    accr_out = rest[idx]; idx += 1
    if with_cnt:
      cntp_out = rest[idx]; idx += 1
    if with_r:
      r_out = rest[idx]; idx += 1
    (pkv0, pkv1, wrv, rA, rB, rbuf0, rbuf1, accr_sh, semA,
     semB) = rest[idx:idx + 10]
    idx += 10
    pkvs = (pkv0, pkv1)
    rbufs = (rbuf0, rbuf1)
    if with_cnt:
      cacc = rest[idx]; idx += 1

    c = lax.axis_index("c")
    s = lax.axis_index("s")
    wid = _wid()
    zf = jnp.zeros((16,), jnp.float32)

    def zbody(i, _):
      for g in range(H // 16):
        rbuf0[i, pl.ds(g * 16, 16)] = zf
      return 0
    lax.fori_loop(0, ECH, zbody, 0)
    for t in range(STRIPE // ECH):
      pltpu.sync_copy(rbuf0, accr_sh.at[pl.ds(s * STRIPE + t * ECH, ECH)])
    if with_cnt:
      def cz(i, _):
        cacc[pl.ds(i * 16, 16)] = zf
        return 0
      lax.fori_loop(0, NP * 2 // 16, cz, 0)
    plsc.subcore_barrier()

    pltpu.sync_copy(wr_hbm, wrv)
    wr_parts = [wrv[pl.ds(g * 16, 16)] for g in range(H // 16)]
    zeros16 = lax.iota(jnp.int32, 16) * 0
    onesf = zeros16.astype(jnp.float32) + 1.0

    twos16 = zeros16 + 2

    def scatter_chunk(bp, jprev):
      pltpu.sync_copy(rbufs[bp], accr_sh.at[pkvs[bp].at[0]], add=True)
      if with_cnt:
        for g in range(ECH // 16):
          plsc.addupdate_scatter(
              cacc, [pkvs[bp][0, pl.ds(g * 16, 16)] * 2], onesf)
      if with_r:
        base_p = wid * EW + jprev * ECH
        pltpu.sync_copy(rbufs[bp], r_out.at[pl.ds(base_p, ECH)])

    def compute_chunk(b):
      pkv = pkvs[b]
      rbuf = rbufs[b]

      @plsc.parallel_loop(0, ECH, 1, unroll=2)
      def ebody(e):
        de = plsc.bitcast(
            plsc.load_gather(pkv, [twos16, zeros16 + e]), jnp.float32)
        for g in range(H // 16):
          sl = pl.ds(g * 16, 16)
          val = rA[e, sl] + rB[e, sl] + de * wr_parts[g]
          rbuf[e, sl] = jnp.maximum(val, 0.0)

    if True:
      def chunk(j, _):
        gc = wid * ENCHUNK + j
        pltpu.sync_copy(pk_hbm.at[gc], pkv0)
        cpA = pltpu.async_copy(a_hbm.at[pkv0.at[0]], rA, semA)
        cpB = pltpu.async_copy(b_hbm.at[pkv0.at[1]], rB, semB)
        cpA.wait()
        cpB.wait()
        compute_chunk(0)
        scatter_chunk(0, j)
        plsc.subcore_barrier()
        return 0
      lax.fori_loop(0, ENCHUNK, chunk, 0)
    else:
      def chunk(jj, _):
        for b in (0, 1):
          j = 2 * jj + b
          gc = wid * ENCHUNK + j
          pltpu.sync_copy(pk_hbm.at[gc], pkvs[b])
          cpA = pltpu.async_copy(a_hbm.at[pkvs[b].at[0]], rA, semA)
          cpB = pltpu.async_copy(b_hbm.at[pkvs[b].at[1]], rB, semB)
          if b == 0:
            @pl.when(jj > 0)
            def _():
              scatter_chunk(1, j - 1)
          else:
            scatter_chunk(0, j - 1)
          plsc.subcore_barrier()
          cpA.wait()
          cpB.wait()
          compute_chunk(b)
        return 0
      lax.fori_loop(0, ENCHUNK // 2, chunk, 0)
      scatter_chunk(1, ENCHUNK - 1)

    plsc.subcore_barrier()

    def cpout(t, _):
      sl2 = pl.ds(s * STRIPE + t * ECH, ECH)
      pltpu.sync_copy(accr_sh.at[sl2], accr_out.at[c, sl2])
      return 0
    lax.fori_loop(0, STRIPE // ECH, cpout, 0)
    if with_cnt:
      pltpu.sync_copy(cacc, cntp_out.at[wid])

  return pl.kernel(body, out_type=tuple(outs), mesh=_MESH,
                   compiler_params=_SC_PARAMS, scratch_types=scratch)


_sc_edge_first = _make_sc_edge(True, False)
_sc_edge_mid = _make_sc_edge(False, False)
_sc_edge_last = _make_sc_edge(False, True)


@functools.partial(
    pl.kernel,
    out_type=jax.ShapeDtypeStruct((NW, NP * 2), jnp.float32),
    mesh=_MESH,
    compiler_params=_SC_PARAMS,
    scratch_types=[
        pltpu.VMEM((CH,), jnp.int32),
        pltpu.VMEM((CH, 16), jnp.float32),
        pltpu.VMEM((NP * 2,), jnp.float32),
    ],
)
def _sc_srows(srows_hbm, row_hbm, sacc_out, rowv, srcv, acc):
  wid = _wid()
  zf = jnp.zeros((16,), jnp.float32)
  iota16 = lax.iota(jnp.int32, 16)
  zeros16 = iota16 * 0
  ones16 = zeros16 + 1

  def zb(i, _):
    acc[pl.ds(i * 16, 16)] = zf
    return 0
  lax.fori_loop(0, NP * 2 // 16, zb, 0)

  def chunk(j, _):
    base = wid * EW + j * CH
    pltpu.sync_copy(row_hbm.at[pl.ds(base, CH)], rowv)
    pltpu.sync_copy(srows_hbm.at[pl.ds(base, CH)], srcv)
    for g in range(CH // 16):
      sl = pl.ds(g * 16, 16)
      ir = rowv[sl]
      rows16 = zeros16 + g * 16 + iota16
      v0 = plsc.load_gather(srcv, [rows16, zeros16])
      v1 = plsc.load_gather(srcv, [rows16, ones16])
      plsc.addupdate_scatter(acc, [ir * 2], v0)
      plsc.addupdate_scatter(acc, [ir * 2 + 1], v1)
    return 0
  lax.fori_loop(0, NCHUNK, chunk, 0)
  pltpu.sync_copy(acc, sacc_out.at[wid])


def _dot(a, b):
  return jnp.dot(a, b, preferred_element_type=jnp.float32)


_full128 = pl.BlockSpec((H, H), lambda i: (0, 0))
_bias = pl.BlockSpec((1, H), lambda i: (0, 0))
_nodeblk = pl.BlockSpec((NBLK, H), lambda i: (i, 0))
_accblk = pl.BlockSpec((NC, NBLK, H), lambda i: (0, i, 0))
_cntblk = pl.BlockSpec((NBLK, 2), lambda i: (i, 0))
_saccblk = pl.BlockSpec((NBLK, 2), lambda i: (i, 0))


def _tc_pre_body(h_ref, wemb_ref, bemb_ref, wa_ref, ba_ref, wb_ref,
                 h0_ref, a_ref, b_ref):
  h0 = _dot(h_ref[...], wemb_ref[...]) + bemb_ref[...]
  h0_ref[...] = h0
  a_ref[...] = _dot(h0, wa_ref[...]) + ba_ref[...]
  b_ref[...] = _dot(h0, wb_ref[...])


_tc_pre = pl.pallas_call(
    _tc_pre_body,
    grid=(NP // NBLK,),
    in_specs=[_nodeblk, _full128, _bias, _full128, _bias, _full128],
    out_specs=[_nodeblk, _nodeblk, _nodeblk],
    out_shape=[jax.ShapeDtypeStruct((NP, H), jnp.float32)] * 3,
)


def _make_tc_node(first, last):
  def body(*refs):
    idx = 0
    h_ref = refs[idx]; idx += 1
    accr_ref = refs[idx]; idx += 1
    cnt_ref = refs[idx]; idx += 1
    we2, be2, wn1a, wn1b, bn1, wn2, bn2 = refs[idx:idx + 7]; idx += 7
    if last:
      wv1, bv1, wv2, bv2 = refs[idx:idx + 4]; idx += 4
    else:
      wa, ba, wb = refs[idx:idx + 3]; idx += 3
    hn_ref = refs[idx]; idx += 1
    if last:
      vpre_ref = refs[idx]; idx += 1
    else:
      a_ref, b_ref = refs[idx:idx + 2]; idx += 2
    cnt = cnt_ref[...][:, 0:1]
    h = h_ref[...]
    seg = accr_ref[0] + accr_ref[1]
    m = _dot(seg, we2[...]) + cnt * be2[...]
    u = jax.nn.relu(_dot(h, wn1a[...]) + _dot(m, wn1b[...]) + bn1[...])
    hn = h + _dot(u, wn2[...]) + bn2[...]
    hn_ref[...] = hn
    if last:
      vp = jax.nn.relu(_dot(hn, wv1[...]) + bv1[...])
      vpre_ref[...] = _dot(vp, wv2[...]) + bv2[...]
    else:
      a_ref[...] = _dot(hn, wa[...]) + ba[...]
      b_ref[...] = _dot(hn, wb[...])

  in_specs = [_nodeblk, _accblk, _cntblk,
              _full128, _bias, _full128, _full128, _bias, _full128, _bias]
  if last:
    in_specs += [_full128, _bias, _full128, _bias]
  else:
    in_specs += [_full128, _bias, _full128]
  out_specs = [_nodeblk]
  out_shape = [jax.ShapeDtypeStruct((NP, H), jnp.float32)]
  if last:
    out_specs += [_nodeblk]
    out_shape += [jax.ShapeDtypeStruct((NP, H), jnp.float32)]
  else:
    out_specs += [_nodeblk, _nodeblk]
    out_shape += [jax.ShapeDtypeStruct((NP, H), jnp.float32)] * 2
  return pl.pallas_call(body, grid=(NP // NBLK,), in_specs=in_specs,
                        out_specs=out_specs, out_shape=out_shape)


_tc_node_mid = _make_tc_node(False, False)
_tc_node_last = _make_tc_node(False, True)


def _tc_red32_body(x_ref, o_ref):
  o_ref[...] = jnp.sum(x_ref[...], axis=0)


_tc_red32 = pl.pallas_call(
    _tc_red32_body,
    grid=(1,),
    in_specs=[pl.BlockSpec((NW, NP * 2), lambda i: (0, 0))],
    out_specs=pl.BlockSpec((NP * 2,), lambda i: (0,)),
    out_shape=jax.ShapeDtypeStruct((NP * 2,), jnp.float32),
)


def _tc_gate_body(r_ref, rp_ref, we2_ref, wc1_ref, bec_ref, wc2_ref, bc2_ref,
                  srows_ref):
  wec = _dot(we2_ref[...], wc1_ref[...])
  t = jax.nn.relu(_dot(r_ref[...], wec) + bec_ref[...])
  c = jnp.sum(t * wc2_ref[...], axis=-1, keepdims=True) + bc2_ref[:, 0:1]
  srows_ref[...] = rp_ref[...] * c


_tc_gate = pl.pallas_call(
    _tc_gate_body,
    grid=(EP // EBLK,),
    in_specs=[pl.BlockSpec((EBLK, H), lambda i: (i, 0)),
              pl.BlockSpec((EBLK, 16), lambda i: (i, 0)),
              _full128, _full128, _bias, _bias, _bias],
    out_specs=pl.BlockSpec((EBLK, 16), lambda i: (i, 0)),
    out_shape=jax.ShapeDtypeStruct((EP, 16), jnp.float32),
)


def _tc_final_body(vpre_ref, sacc_ref, cnt_ref, v_ref):
  s2 = sacc_ref[...]
  cnt = cnt_ref[...][:, 0:1]
  v2 = vpre_ref[...][:, 0:2] + s2 / jnp.maximum(cnt, 1.0)
  nrm = jnp.sqrt(jnp.sum(v2 * v2, axis=-1, keepdims=True))
  v2 = v2 / jnp.maximum(nrm, 1e-12)
  v_ref[...] = jnp.concatenate(
      [v2, jnp.zeros((v2.shape[0], H - 2), jnp.float32)], axis=-1)


_tc_final = pl.pallas_call(
    _tc_final_body,
    grid=(NP // NBLK,),
    in_specs=[_nodeblk, _saccblk, _cntblk],
    out_specs=_nodeblk,
    out_shape=jax.ShapeDtypeStruct((NP, H), jnp.float32),
)


def kernel(h, x, edge_index, params):
  p = params
  f32 = jnp.float32
  hp = jnp.pad(h.astype(f32), ((0, NP - N), (0, 0)))
  x0 = jnp.pad(x[:, 0].astype(f32), (0, NP - N))
  x1 = jnp.pad(x[:, 1].astype(f32), (0, NP - N))
  row = jnp.pad(edge_index[0], (0, EP - E), constant_values=N)
  col = jnp.pad(edge_index[1], (0, EP - E), constant_values=N)

  we1a = p['W_e1'][:H]
  we1b = p['W_e1'][H:2 * H]
  wr = p['W_e1'][2 * H]
  bemb = p['b_emb'].reshape(1, H)
  be1 = p['b_e1'].reshape(1, H)
  be2 = p['b_e2'].reshape(1, H)
  wn1a = p['W_n1'][:H]
  wn1b = p['W_n1'][H:2 * H]
  bn1 = p['b_n1'].reshape(1, H)
  bn2 = p['b_n2'].reshape(1, H)
  bec = (p['b_e2'] @ p['W_c1'] + p['b_c1']).reshape(1, H)
  wc2 = p['W_c2'][:, 0].reshape(1, H)
  bc2 = jnp.broadcast_to(p['b_c2'].reshape(1, 1), (1, H))
  wv2 = jnp.pad(p['W_v2'], ((0, 0), (0, H - 2)))
  bv2 = jnp.pad(p['b_v2'], (0, H - 2)).reshape(1, H)

  d, rp16 = _sc_geom(x0, x1, row, col)
  pk = jnp.stack([row.reshape(EP // ECH, ECH),
                  col.reshape(EP // ECH, ECH),
                  d.reshape(EP // ECH, ECH).view(jnp.int32)], axis=1)
  h0, a0, b0 = _tc_pre(hp, p['W_emb'], bemb, we1a, be1, we1b)

  accr, cntp = _sc_edge_first(a0, b0, pk, wr)
  cnt2 = _tc_red32(cntp).reshape(NP, 2)
  h1, a1, b1 = _tc_node_mid(
      h0, accr, cnt2, p['W_e2'], be2, wn1a, wn1b, bn1, p['W_n2'], bn2,
      we1a, be1, we1b)

  accr, = _sc_edge_mid(a1, b1, pk, wr)
  h2, a2, b2 = _tc_node_mid(
      h1, accr, cnt2, p['W_e2'], be2, wn1a, wn1b, bn1, p['W_n2'], bn2,
      we1a, be1, we1b)

  accr, = _sc_edge_mid(a2, b2, pk, wr)
  h3, a3, b3 = _tc_node_mid(
      h2, accr, cnt2, p['W_e2'], be2, wn1a, wn1b, bn1, p['W_n2'], bn2,
      we1a, be1, we1b)

  accr, r = _sc_edge_last(a3, b3, pk, wr)
  h4, vpre = _tc_node_last(
      h3, accr, cnt2, p['W_e2'], be2, wn1a, wn1b, bn1, p['W_n2'], bn2,
      p['W_v1'], p['b_v1'].reshape(1, H), wv2, bv2)

  srows = _tc_gate(r, rp16, p['W_e2'], p['W_c1'], bec, wc2, bc2)
  sacc = _sc_srows(srows, row)
  s2 = _tc_red32(sacc).reshape(NP, 2)
  vfull = _tc_final(vpre, s2, cnt2)

  return (h4[:N], x, vfull[:N, :2])
